# baseline (speedup 1.0000x reference)
import functools

import jax
import jax.numpy as jnp
from jax import lax
from jax.experimental import pallas as pl
from jax.experimental.pallas import tpu as pltpu
from jax.experimental.pallas import tpu_sc as plsc

NUM_CORES = 2
NUM_SUBCORES = 16
NUM_WORKERS = NUM_CORES * NUM_SUBCORES
L = 16


def _make_gather(batch: int, seq: int, dim: int):
    assert batch % (NUM_WORKERS * 256) == 0 and dim == 64 and seq % 2 == 0
    bt_per_w = (batch // 128) // NUM_WORKERS
    n_half = bt_per_w // 2
    n_virt = n_half * seq

    mesh = plsc.VectorSubcoreMesh(
        core_axis_name="c", subcore_axis_name="s",
        num_cores=NUM_CORES, num_subcores=NUM_SUBCORES)

    @functools.partial(
        pl.kernel,
        out_type=jax.ShapeDtypeStruct((seq, dim // 8, batch // 128, 8, 128),
                                      jnp.float32),
        mesh=mesh,
        compiler_params=pltpu.CompilerParams(
            use_tc_tiling_on_sc=False, needs_layout_passes=False),
        scratch_types=[
            pltpu.VMEM((256, seq), jnp.int32),
            pltpu.VMEM((n_half, seq, 2, 128), jnp.int32),
            pltpu.VMEM((2, 256, dim), jnp.float32),
            pltpu.VMEM((2, 8, 2, 8, 128), jnp.float32),
            pltpu.SemaphoreType.DMA,
            pltpu.SemaphoreType.DMA,
            pltpu.SemaphoreType.DMA,
            pltpu.SemaphoreType.DMA,
        ],
    )
    def gather_kernel(idx_hbm, table_hbm, z_hbm, xstage, idxt, rows, zbuf,
                      gsem0, gsem1, osem0, osem1):
        wid = lax.axis_index("s") * NUM_CORES + lax.axis_index("c")
        bt0 = wid * bt_per_w

        iota = lax.iota(jnp.int32, L)
        zer = jnp.full((L,), 0, jnp.int32)
        row_base = [p * 128 + 16 * g + iota
                    for p in range(2) for g in range(128 // L)]

        for h in range(n_half):
            pltpu.sync_copy(
                idx_hbm.at[pl.ds((bt0 + 2 * h) * 128, 256)], xstage)
            for p in range(2):
                def tbody(s, _, p=p, h=h):
                    for g in range(128 // L):
                        v = plsc.load_gather(
                            xstage,
                            [p * 128 + 16 * g + iota,
                             jnp.full((L,), 0, jnp.int32) + s])
                        idxt[h, s, p, pl.ds(16 * g, L)] = v
                    return 0
                lax.fori_loop(0, seq, tbody, 0)

        def hs(v):
            return v // seq, lax.rem(v, seq)

        def fire(v, par, gsem):
            h, s = hs(v)
            for p in range(2):
                pltpu.async_copy(
                    table_hbm.at[idxt.at[h, s, p]],
                    rows.at[par, pl.ds(p * 128, 128)], gsem)

        def drain_gathers(par, gsem):
            pltpu.make_async_copy(
                table_hbm.at[pl.ds(0, 256)], rows.at[par], gsem).wait()

        def transform(par):
            src = rows.at[par]
            dst = zbuf.at[par]
            for p in range(2):
                for e in range(dim):
                    cole = zer + e
                    for g in range(128 // L):
                        v = plsc.load_gather(src, [row_base[p * 8 + g], cole])
                        dst[e // 8, p, e % 8, pl.ds(16 * g, L)] = v

        def fire_out(v, par, osem):
            h, s = hs(v)
            for ehi in range(8):
                pltpu.async_copy(
                    zbuf.at[par, ehi],
                    z_hbm.at[s, ehi, pl.ds(bt0 + 2 * h, 2)], osem)

        def drain_out(par, osem):
            for ehi in range(8):
                pltpu.make_async_copy(
                    z_hbm.at[0, 0, pl.ds(0, 2)], zbuf.at[par, ehi],
                    osem).wait()

        fire(0, 0, gsem0)

        def body(q, _):
            v = 2 * q
            fire(v + 1, 1, gsem1)
            drain_gathers(0, gsem0)

            @pl.when(q > 0)
            def _():
                drain_out(0, osem0)

            transform(0)
            fire_out(v, 0, osem0)

            @pl.when(q < n_virt // 2 - 1)
            def _():
                fire(v + 2, 0, gsem0)

            drain_gathers(1, gsem1)

            @pl.when(q > 0)
            def _():
                drain_out(1, osem1)

            transform(1)
            fire_out(v + 1, 1, osem1)
            return 0

        lax.fori_loop(0, n_virt // 2, body, 0)
        drain_out(0, osem0)
        drain_out(1, osem1)

    return gather_kernel


def kernel(x, embedding):
    b, s = x.shape
    d = embedding.shape[1]
    z = _make_gather(b, s, d)(x.astype(jnp.int32), embedding)
    return z.transpose(2, 4, 0, 1, 3).reshape(b, s, d)


# speedup vs baseline: 1.5478x; 1.5478x over previous
import functools

import jax
import jax.numpy as jnp
from jax import lax
from jax.experimental import pallas as pl
from jax.experimental.pallas import tpu as pltpu
from jax.experimental.pallas import tpu_sc as plsc

NUM_CORES = 2
NUM_SUBCORES = 16
NUM_WORKERS = NUM_CORES * NUM_SUBCORES
L = 16


def _make_gather(batch: int, seq: int, dim: int):
    assert batch % (NUM_WORKERS * 256) == 0 and dim == 64 and seq % 2 == 0
    bt_per_w = (batch // 128) // NUM_WORKERS
    n_half = bt_per_w // 2
    n_virt = n_half * seq

    mesh = plsc.VectorSubcoreMesh(
        core_axis_name="c", subcore_axis_name="s",
        num_cores=NUM_CORES, num_subcores=NUM_SUBCORES)

    @functools.partial(
        pl.kernel,
        out_type=jax.ShapeDtypeStruct((seq, dim // 8, batch // 128, 8, 128),
                                      jnp.float32),
        mesh=mesh,
        compiler_params=pltpu.CompilerParams(
            use_tc_tiling_on_sc=False, needs_layout_passes=False),
        scratch_types=[
            pltpu.VMEM((256, seq), jnp.int32),
            pltpu.VMEM((n_half, seq, 2, 128), jnp.int32),
            pltpu.VMEM((2, 256, dim), jnp.float32),
            pltpu.VMEM((2, 2, dim, 128), jnp.float32),
            pltpu.SemaphoreType.DMA,
            pltpu.SemaphoreType.DMA,
            pltpu.SemaphoreType.DMA,
            pltpu.SemaphoreType.DMA,
        ],
    )
    def gather_kernel(idx_hbm, table_hbm, z_hbm, xstage, idxt, rows, zbuf,
                      gsem0, gsem1, osem0, osem1):
        wid = lax.axis_index("s") * NUM_CORES + lax.axis_index("c")
        bt0 = wid * bt_per_w

        iota = lax.iota(jnp.int32, L)
        zer = jnp.full((L,), 0, jnp.int32)
        row_base = [p * 128 + 16 * g + iota
                    for p in range(2) for g in range(128 // L)]

        for h in range(n_half):
            pltpu.sync_copy(
                idx_hbm.at[pl.ds((bt0 + 2 * h) * 128, 256)], xstage)
            for p in range(2):
                def tbody(s, _, p=p, h=h):
                    for g in range(128 // L):
                        v = plsc.load_gather(
                            xstage,
                            [p * 128 + 16 * g + iota,
                             jnp.full((L,), 0, jnp.int32) + s])
                        idxt[h, s, p, pl.ds(16 * g, L)] = v
                    return 0
                lax.fori_loop(0, seq, tbody, 0)

        def hs(v):
            return v // seq, lax.rem(v, seq)

        def fire(v, par, gsem):
            h, s = hs(v)
            for p in range(2):
                pltpu.async_copy(
                    table_hbm.at[idxt.at[h, s, p]],
                    rows.at[par, pl.ds(p * 128, 128)], gsem)

        def drain_gathers(par, gsem):
            pltpu.make_async_copy(
                table_hbm.at[pl.ds(0, 256)], rows.at[par], gsem).wait()

        def transform(par):
            src = rows.at[par]
            for p in range(2):
                dst = zbuf.at[par, p]

                @plsc.parallel_loop(0, dim, unroll=8)
                def _(e, p=p, dst=dst):
                    cole = zer + e
                    for g in range(128 // L):
                        v = plsc.load_gather(src, [row_base[p * 8 + g], cole])
                        dst[e, pl.ds(16 * g, L)] = v

        def fire_out(v, par, osem):
            h, s = hs(v)
            for ehi in range(8):
                pltpu.async_copy(
                    zbuf.at[par, :, pl.ds(8 * ehi, 8)],
                    z_hbm.at[s, ehi, pl.ds(bt0 + 2 * h, 2)], osem)

        def drain_out(par, osem):
            for ehi in range(8):
                pltpu.make_async_copy(
                    z_hbm.at[0, 0, pl.ds(0, 2)],
                    zbuf.at[par, :, pl.ds(8 * ehi, 8)], osem).wait()

        fire(0, 0, gsem0)

        def body(q, _):
            v = 2 * q
            fire(v + 1, 1, gsem1)
            drain_gathers(0, gsem0)

            @pl.when(q > 0)
            def _():
                drain_out(0, osem0)

            transform(0)
            fire_out(v, 0, osem0)

            @pl.when(q < n_virt // 2 - 1)
            def _():
                fire(v + 2, 0, gsem0)

            drain_gathers(1, gsem1)

            @pl.when(q > 0)
            def _():
                drain_out(1, osem1)

            transform(1)
            fire_out(v + 1, 1, osem1)
            return 0

        lax.fori_loop(0, n_virt // 2, body, 0)
        drain_out(0, osem0)
        drain_out(1, osem1)

    return gather_kernel


def kernel(x, embedding):
    b, s = x.shape
    d = embedding.shape[1]
    z = _make_gather(b, s, d)(x.astype(jnp.int32), embedding)
    return z.transpose(2, 4, 0, 1, 3).reshape(b, s, d)


# speedup vs baseline: 1.6105x; 1.0405x over previous
import functools

import jax
import jax.numpy as jnp
from jax import lax
from jax.experimental import pallas as pl
from jax.experimental.pallas import tpu as pltpu
from jax.experimental.pallas import tpu_sc as plsc

NUM_CORES = 2
NUM_SUBCORES = 16
NUM_WORKERS = NUM_CORES * NUM_SUBCORES
L = 16


def _make_gather(batch: int, seq: int, dim: int):
    assert batch % (NUM_WORKERS * 256) == 0 and dim == 64 and seq % 2 == 0
    bt_per_w = (batch // 128) // NUM_WORKERS
    n_half = bt_per_w // 2
    n_virt = n_half * seq

    mesh = plsc.VectorSubcoreMesh(
        core_axis_name="c", subcore_axis_name="s",
        num_cores=NUM_CORES, num_subcores=NUM_SUBCORES)

    @functools.partial(
        pl.kernel,
        out_type=jax.ShapeDtypeStruct((seq, dim // 8, batch // 128, 8, 128),
                                      jnp.float32),
        mesh=mesh,
        compiler_params=pltpu.CompilerParams(
            use_tc_tiling_on_sc=False, needs_layout_passes=False),
        scratch_types=[
            pltpu.VMEM((256, seq), jnp.int32),
            pltpu.VMEM((n_half, seq, 2, 128), jnp.int32),
            pltpu.VMEM((2, 256, dim), jnp.float32),
            pltpu.VMEM((2, 2, dim, 128), jnp.float32),
            pltpu.SemaphoreType.DMA,
            pltpu.SemaphoreType.DMA,
            pltpu.SemaphoreType.DMA,
            pltpu.SemaphoreType.DMA,
        ],
    )
    def gather_kernel(idx_hbm, table_hbm, z_hbm, xstage, idxt, rows, zbuf,
                      gsem0, gsem1, osem0, osem1):
        wid = lax.axis_index("s") * NUM_CORES + lax.axis_index("c")
        bt0 = wid * bt_per_w

        iota = lax.iota(jnp.int32, L)
        zer = jnp.full((L,), 0, jnp.int32)
        row_base = [p * 128 + 16 * g + iota
                    for p in range(2) for g in range(128 // L)]

        for h in range(n_half):
            pltpu.sync_copy(
                idx_hbm.at[pl.ds((bt0 + 2 * h) * 128, 256)], xstage)
            for p in range(2):
                def tbody(s, _, p=p, h=h):
                    for g in range(128 // L):
                        v = plsc.load_gather(
                            xstage,
                            [p * 128 + 16 * g + iota,
                             jnp.full((L,), 0, jnp.int32) + s])
                        idxt[h, s, p, pl.ds(16 * g, L)] = v
                    return 0
                lax.fori_loop(0, seq, tbody, 0)

        def hs(v):
            return v // seq, lax.rem(v, seq)

        def fire(v, par, gsem):
            h, s = hs(v)
            for p in range(2):
                pltpu.async_copy(
                    table_hbm.at[idxt.at[h, s, p]],
                    rows.at[par, pl.ds(p * 128, 128)], gsem)

        def drain_gathers(par, gsem):
            pltpu.make_async_copy(
                table_hbm.at[pl.ds(0, 256)], rows.at[par], gsem).wait()

        def transform(par):
            src = rows.at[par]
            for p in range(2):
                dst = zbuf.at[par, p]

                @plsc.parallel_loop(0, dim, unroll=4)
                def _(e, p=p, dst=dst):
                    cole = zer + e
                    for g in range(128 // L):
                        v = plsc.load_gather(src, [row_base[p * 8 + g], cole])
                        dst[e, pl.ds(16 * g, L)] = v

        def fire_out(v, par, osem):
            h, s = hs(v)
            for ehi in range(8):
                pltpu.async_copy(
                    zbuf.at[par, :, pl.ds(8 * ehi, 8)],
                    z_hbm.at[s, ehi, pl.ds(bt0 + 2 * h, 2)], osem)

        def drain_out(par, osem):
            for ehi in range(8):
                pltpu.make_async_copy(
                    z_hbm.at[0, 0, pl.ds(0, 2)],
                    zbuf.at[par, :, pl.ds(8 * ehi, 8)], osem).wait()

        fire(0, 0, gsem0)

        def body(q, _):
            v = 2 * q
            fire(v + 1, 1, gsem1)
            drain_gathers(0, gsem0)

            @pl.when(q > 0)
            def _():
                drain_out(0, osem0)

            transform(0)
            fire_out(v, 0, osem0)

            @pl.when(q < n_virt // 2 - 1)
            def _():
                fire(v + 2, 0, gsem0)

            drain_gathers(1, gsem1)

            @pl.when(q > 0)
            def _():
                drain_out(1, osem1)

            transform(1)
            fire_out(v + 1, 1, osem1)
            return 0

        lax.fori_loop(0, n_virt // 2, body, 0)
        drain_out(0, osem0)
        drain_out(1, osem1)

    return gather_kernel


def kernel(x, embedding):
    b, s = x.shape
    d = embedding.shape[1]
    z = _make_gather(b, s, d)(x.astype(jnp.int32), embedding)
    return z.transpose(2, 4, 0, 1, 3).reshape(b, s, d)


# speedup vs baseline: 1.6297x; 1.0119x over previous
import functools

import jax
import jax.numpy as jnp
from jax import lax
from jax.experimental import pallas as pl
from jax.experimental.pallas import tpu as pltpu
from jax.experimental.pallas import tpu_sc as plsc

NUM_CORES = 2
NUM_SUBCORES = 16
NUM_WORKERS = NUM_CORES * NUM_SUBCORES
CHUNK = 128
GK = 4


def _make_gather(total_rows: int, dim: int):
    assert total_rows % (NUM_WORKERS * CHUNK * GK * 2) == 0
    rows_per_w = total_rows // NUM_WORKERS
    chunks_per_w = rows_per_w // CHUNK
    group_rows = GK * CHUNK
    num_pairs = chunks_per_w // (2 * GK)

    mesh = plsc.VectorSubcoreMesh(
        core_axis_name="c", subcore_axis_name="s",
        num_cores=NUM_CORES, num_subcores=NUM_SUBCORES)

    @functools.partial(
        pl.kernel,
        out_type=jax.ShapeDtypeStruct((total_rows, dim), jnp.float32),
        mesh=mesh,
        compiler_params=pltpu.CompilerParams(use_tc_tiling_on_sc=False),
        scratch_types=[
            pltpu.VMEM((chunks_per_w, CHUNK), jnp.int32),
            pltpu.VMEM((group_rows, dim), jnp.float32),
            pltpu.VMEM((group_rows, dim), jnp.float32),
            pltpu.SemaphoreType.DMA,
            pltpu.SemaphoreType.DMA,
            pltpu.SemaphoreType.DMA,
        ],
    )
    def gather_kernel(idx_hbm, table_hbm, out_hbm, idx_v, rows_a, rows_b,
                      gsem_a, gsem_b, osem):
        wid = lax.axis_index("s") * NUM_CORES + lax.axis_index("c")
        chunk_base = wid * chunks_per_w
        row_base = wid * rows_per_w

        pltpu.sync_copy(idx_hbm.at[pl.ds(chunk_base, chunks_per_w)], idx_v)

        def fire(group, buf, sem):
            for j in range(GK):
                pltpu.async_copy(
                    table_hbm.at[idx_v.at[group * GK + j]],
                    buf.at[pl.ds(j * CHUNK, CHUNK)], sem)

        def drain_gathers(buf, sem):
            pltpu.make_async_copy(
                out_hbm.at[pl.ds(0, group_rows)], buf, sem).wait()

        def scatter(group, buf):
            pltpu.async_copy(
                buf, out_hbm.at[pl.ds(row_base + group * group_rows,
                                      group_rows)], osem).wait()

        fire(0, rows_a, gsem_a)

        def body(q, _):
            a = 2 * q
            fire(a + 1, rows_b, gsem_b)
            drain_gathers(rows_a, gsem_a)
            scatter(a, rows_a)

            @pl.when(q < num_pairs - 1)
            def _():
                fire(a + 2, rows_a, gsem_a)

            drain_gathers(rows_b, gsem_b)
            scatter(a + 1, rows_b)
            return 0

        lax.fori_loop(0, num_pairs, body, 0)

    return gather_kernel


def kernel(x, embedding):
    b, s = x.shape
    total = b * s
    idx2d = x.reshape(total // CHUNK, CHUNK).astype(jnp.int32)
    out = _make_gather(total, embedding.shape[1])(idx2d, embedding)
    return out.reshape(b, s, embedding.shape[1])
